# baseline (speedup 1.0000x reference)
import functools

import jax
import jax.numpy as jnp
from jax import lax
from jax.experimental import pallas as pl
from jax.experimental.pallas import tpu as pltpu
from jax.experimental.pallas import tpu_sc as plsc

_CHUNK = 64
_NB = 2
_GRP = _NB * _CHUNK


def _sc_spmm(xtab, rows_p, cols_p, vals_p, zeros_h, *, m, fin, ntiles,
             ncores, ngrp):
    rpt = m // ntiles
    g3 = ngrp + 2

    mesh = plsc.VectorSubcoreMesh(core_axis_name="c", subcore_axis_name="s")

    def body(xtab_h, rows_h, cols_h, vals_h, zeros_hbm, ycat_h, *scr):
        c = lax.axis_index("c")
        s = lax.axis_index("s")
        acc, rowst, colst, valst = scr[:4]
        gbuf = scr[4:4 + _NB]
        sbuf = scr[4 + _NB:4 + 2 * _NB]
        isem = scr[4 + 2 * _NB:7 + 2 * _NB]
        gsem = scr[7 + 2 * _NB:7 + 3 * _NB]
        ssem = scr[7 + 3 * _NB:7 + 4 * _NB]
        wbase = (c * ntiles + s) * g3

        def issue_idx(g, slot, sem):
            pltpu.async_copy(rows_h.at[wbase + g], rowst.at[slot], sem)
            pltpu.async_copy(cols_h.at[wbase + g], colst.at[slot], sem)
            pltpu.async_copy(vals_h.at[wbase + g], valst.at[slot], sem)

        def wait_idx(g, slot, sem):
            pltpu.make_async_copy(rows_h.at[wbase + g], rowst.at[slot],
                                  sem).wait()
            pltpu.make_async_copy(cols_h.at[wbase + g], colst.at[slot],
                                  sem).wait()
            pltpu.make_async_copy(vals_h.at[wbase + g], valst.at[slot],
                                  sem).wait()

        pltpu.sync_copy(zeros_hbm.at[pl.ds(s * rpt, rpt)],
                        acc.at[pl.ds(s * rpt, rpt)])
        plsc.subcore_barrier()

        issue_idx(0, 0, isem[0])
        wait_idx(0, 0, isem[0])
        issue_idx(1, 1, isem[1])
        for b in range(_NB):
            pltpu.sync_copy(zeros_hbm.at[pl.ds(0, _CHUNK)], sbuf[b])
            pltpu.async_copy(sbuf[b], acc.at[rowst.at[0, b]], ssem[b],
                             add=True)
            pltpu.async_copy(xtab_h.at[colst.at[0, b]], gbuf[b], gsem[b])

        def macro(t, carry):
            for p in range(3):
                g = t * 3 + p
                nslot = (p + 1) % 3
                xslot = (p + 2) % 3
                wait_idx(g + 1, nslot, isem[nslot])
                for b in range(_NB):
                    pltpu.make_async_copy(sbuf[b], acc.at[rowst.at[p, b]],
                                          ssem[b]).wait()
                    pltpu.make_async_copy(xtab_h.at[colst.at[p, b]], gbuf[b],
                                          gsem[b]).wait()

                    def edge(j, carry2):
                        for u in range(4):
                            e = j * 4 + u
                            v16 = valst[p, b * _CHUNK + e]
                            for f in range(fin // 16):
                                sbuf[b][e, pl.ds(f * 16, 16)] = (
                                    gbuf[b][e, pl.ds(f * 16, 16)] * v16)
                        return carry2

                    lax.fori_loop(0, _CHUNK // 4, edge, 0, unroll=2)

                    pltpu.async_copy(sbuf[b], acc.at[rowst.at[p, b]], ssem[b],
                                     add=True)
                    pltpu.async_copy(xtab_h.at[colst.at[nslot, b]], gbuf[b],
                                     gsem[b])
                issue_idx(g + 2, xslot, isem[xslot])
            return carry

        lax.fori_loop(0, ngrp // 3, macro, 0)

        wait_idx(ngrp + 1, 1, isem[1])
        for b in range(_NB):
            pltpu.make_async_copy(sbuf[b], acc.at[rowst.at[2, b]],
                                  ssem[b]).wait()
            pltpu.make_async_copy(xtab_h.at[colst.at[0, b]], gbuf[b],
                                  gsem[b]).wait()
        plsc.subcore_barrier()

        pltpu.sync_copy(acc.at[pl.ds(s * rpt, rpt)],
                        ycat_h.at[pl.ds(c * m + s * rpt, rpt)])

    return pl.kernel(
        body,
        out_type=jax.ShapeDtypeStruct((ncores * m, fin), jnp.float32),
        mesh=mesh,
        scratch_types=[
            pltpu.VMEM_SHARED((m, fin), jnp.float32),
            pltpu.VMEM((3, _NB, _CHUNK), jnp.int32),
            pltpu.VMEM((3, _NB, _CHUNK), jnp.int32),
            pltpu.VMEM((3, _GRP, 16), jnp.float32),
        ] + [pltpu.VMEM((_CHUNK, fin), jnp.float32)] * (2 * _NB)
          + [pltpu.SemaphoreType.DMA] * (3 + 2 * _NB),
        compiler_params=pltpu.CompilerParams(use_tc_tiling_on_sc=False),
    )(xtab, rows_p, cols_p, vals_p, zeros_h)


def _tc_add(yp):
    def body(y_ref, o_ref):
        o_ref[...] = y_ref[0] + y_ref[1]

    return pl.pallas_call(
        body,
        out_shape=jax.ShapeDtypeStruct(yp.shape[1:], jnp.float32),
    )(yp)


def _tc_combine(x0, y1, y2, y3, acat, *, m, fin, fout, bm):
    def body(x0_ref, y1_ref, y2_ref, y3_ref, a_ref, o_ref):
        a = a_ref[...]
        acc = jnp.dot(x0_ref[...], a[0:fin],
                      preferred_element_type=jnp.float32)
        acc += jnp.dot(y1_ref[...], a[fin:2 * fin],
                       preferred_element_type=jnp.float32)
        acc += jnp.dot(y2_ref[...], a[2 * fin:3 * fin],
                       preferred_element_type=jnp.float32)
        a3 = a[3 * fin:4 * fin]
        acc += jnp.dot(y3_ref[0] + y3_ref[1], a3,
                       preferred_element_type=jnp.float32)
        o_ref[...] = acc

    grid = m // bm
    return pl.pallas_call(
        body,
        grid=(grid,),
        in_specs=[
            pl.BlockSpec((bm, fin), lambda i: (i, 0)),
            pl.BlockSpec((bm, fin), lambda i: (i, 0)),
            pl.BlockSpec((bm, fin), lambda i: (i, 0)),
            pl.BlockSpec((2, bm, fin), lambda i: (0, i, 0)),
            pl.BlockSpec((4 * fin, fout), lambda i: (0, 0)),
        ],
        out_specs=pl.BlockSpec((bm, fout), lambda i: (i, 0)),
        out_shape=jax.ShapeDtypeStruct((m, fout), jnp.float32),
    )(x0, y1, y2, y3, acat)


def kernel(input_tensor, kernel, L_rows, L_cols, L_vals):
    b, m, fin = input_tensor.shape
    fout = kernel.shape[1]
    nnz = L_rows.shape[0]

    info = plsc.get_sparse_core_info()
    ncores, ntiles = info.num_cores, info.num_subcores
    nw = ncores * ntiles

    ngrp = -(-nnz // (nw * _GRP))
    ngrp = ((ngrp + 2) // 3) * 3
    g3 = ngrp + 2
    ep = nw * ngrp * _GRP
    pad = ep - nnz
    rows_p = jnp.concatenate([L_rows, jnp.zeros((pad,), jnp.int32)])
    cols_p = jnp.concatenate([L_cols, jnp.zeros((pad,), jnp.int32)])
    vals_p = jnp.concatenate([L_vals, jnp.zeros((pad,), jnp.float32)])

    def to_groups(a):
        a = a.reshape(nw, ngrp * _GRP)
        a = jnp.pad(a, ((0, 0), (0, 2 * _GRP)))
        return a.reshape(nw * g3, _GRP)

    rows_p = to_groups(rows_p).reshape(nw * g3, _NB, _CHUNK)
    cols_p = to_groups(cols_p).reshape(nw * g3, _NB, _CHUNK)
    vals_p = jnp.broadcast_to(
        to_groups(vals_p)[:, :, None], (nw * g3, _GRP, 16)).copy()

    rquantum = ntiles * 8
    mp = ((m + rquantum - 1) // rquantum) * rquantum
    zeros_h = jnp.zeros((mp, fin), jnp.float32)

    x0 = input_tensor[0]
    x0p = jnp.zeros((mp, fin), jnp.float32).at[:m].set(x0)

    spmm = functools.partial(
        _sc_spmm, m=mp, fin=fin, ntiles=ntiles, ncores=ncores, ngrp=ngrp)
    y1p = spmm(x0p, rows_p, cols_p, vals_p, zeros_h)
    y1 = _tc_add(y1p.reshape(2, mp, fin))
    y2p = spmm(y1, rows_p, cols_p, vals_p, zeros_h)
    y2 = _tc_add(y2p.reshape(2, mp, fin))
    y3p = spmm(y2, rows_p, cols_p, vals_p, zeros_h)

    k = kernel.shape[0] // fin - 1
    wr = kernel.reshape(fin, k + 1, fout)
    w0, w1, w2, w3 = wr[:, 0], wr[:, 1], wr[:, 2], wr[:, 3]
    a0 = w0
    a1 = -1.5 * w0 + 1.5 * w1
    a2 = 0.75 * w0 - 1.5 * w1 + 0.75 * w2 + 0.09375 * w3
    a3 = -0.125 * w0 + 0.375 * w1 - 0.375 * w2 - 0.046875 * w3
    acat = jnp.concatenate([a0, a1, a2, a3], axis=0)

    out = _tc_combine(x0, y1, y2, y3p.reshape(2, mp, fin), acat,
                      m=m, fin=fin, fout=fout, bm=1000)
    return out.reshape(b, m, fout)
